# baseline (speedup 1.0000x reference)
import jax
import jax.numpy as jnp
from jax import lax
from jax.experimental import pallas as pl
from jax.experimental.pallas import tpu as pltpu
from jax.experimental.pallas import tpu_sc as plsc

N = 100000
D = 128
G = 512
NC = 2
NS = 16
NW = NC * NS
NLANE = 16
NV = D // NLANE
CHUNK = 256
HALF = CHUNK // 2
NGRP = HALF // NLANE
NSLOT = 196
P = NSLOT * CHUNK
IDXW = 128
TSTEPS = -(-NSLOT // NW) + 1
TCBLK = 512
TCX0 = P // TCBLK
TCGRID = -(-(N - P) // TCBLK)
GROWS = G // NS


def _sc_body(x_hbm, batch_hbm, part_hbm, xb0, xb1, ib0, ib1, fbuf, iref,
             cidx, obuf, zbuf, acc2, accf,
             gs0, gs1, is0, is1, ss0, ss1):
    c = lax.axis_index("c")
    s = lax.axis_index("s")
    w = s * NC + c
    xbufs, ibufs = (xb0, xb1), (ib0, ib1)
    gsems, isems, ssems = (gs0, gs1), (is0, is1), (ss0, ss1)
    lane = lax.broadcasted_iota(jnp.int32, (NLANE,), 0)

    @pl.loop(0, GROWS)
    def _zero_rows(i):
        @pl.loop(0, NV)
        def _zero_lanes(k):
            obuf[i, pl.ds(k * NLANE, NLANE)] = jnp.zeros((NLANE,),
                                                         jnp.float32)

    @pl.loop(0, GROWS * NV)
    def _zero_flat(i):
        zbuf[pl.ds(i * NLANE, NLANE)] = jnp.zeros((NLANE,), jnp.float32)

    pltpu.sync_copy(obuf, acc2.at[pl.ds(s * GROWS, GROWS)])
    pltpu.sync_copy(zbuf, accf.at[pl.ds(s * GROWS * D, GROWS * D)])
    plsc.subcore_barrier()

    def flush(prev, accs):
        base = prev * D + lane
        for k in range(NV):
            fbuf[pl.ds(k * NLANE, NLANE)] = accs[k]
            iref[pl.ds(k * NLANE, NLANE)] = base + k * NLANE
        pltpu.sync_copy(fbuf, accf.at[iref], add=True)

    def issue(slot, b):
        @pl.when(slot < NSLOT)
        def _():
            row0 = pl.multiple_of(slot * CHUNK, CHUNK)
            pltpu.async_copy(x_hbm.at[pl.ds(row0, CHUNK)], xbufs[b],
                             gsems[b])
            pltpu.async_copy(batch_hbm.at[pl.ds(row0, IDXW)],
                             ibufs[b].at[0], isems[b])
            pltpu.async_copy(batch_hbm.at[pl.ds(row0 + IDXW, IDXW)],
                             ibufs[b].at[1], isems[b])

    def process(slot, b):
        @pl.when(slot < NSLOT)
        def _():
            row0 = pl.multiple_of(slot * CHUNK, CHUNK)
            pltpu.make_async_copy(x_hbm.at[pl.ds(row0, CHUNK)], xbufs[b],
                                  gsems[b]).wait()
            pltpu.make_async_copy(batch_hbm.at[pl.ds(row0, IDXW)],
                                  ibufs[b].at[0], isems[b]).wait()
            pltpu.make_async_copy(batch_hbm.at[pl.ds(row0 + IDXW, IDXW)],
                                  ibufs[b].at[1], isems[b]).wait()

            pltpu.async_copy(xbufs[b].at[pl.ds(HALF, HALF)],
                             acc2.at[ibufs[b].at[1]], ssems[b], add=True)

            zero = jnp.zeros((NLANE,), jnp.float32)
            prev0 = ibufs[b][0, pl.ds(0, NLANE)][0]
            init = (prev0,) + (zero,) * NV

            @pl.loop(0, NGRP, init_carry=init)
            def _groups(g, carry):
                prev, accs = carry[0], carry[1:]
                idvec = ibufs[b][0, pl.ds(g * NLANE, NLANE)]
                first, last = idvec[0], idvec[NLANE - 1]
                all_same = jnp.logical_and(first == prev, last == prev)
                gsum = [None] * NV
                for j in range(NLANE):
                    for k in range(NV):
                        v = xbufs[b][g * NLANE + j, pl.ds(k * NLANE, NLANE)]
                        gsum[k] = v if j == 0 else gsum[k] + v

                @pl.when(jnp.logical_not(all_same))
                def _slow():
                    flush(prev, accs)
                    cidx[...] = idvec
                    pltpu.sync_copy(xbufs[b].at[pl.ds(g * NLANE, NLANE)],
                                    acc2.at[cidx], add=True)

                new_prev = jnp.where(all_same, prev, last)
                new_accs = tuple(
                    jnp.where(all_same, a + gs, zero)
                    for a, gs in zip(accs, gsum))
                return (new_prev,) + new_accs

            flush(_groups[0], _groups[1:])

            pltpu.make_async_copy(xbufs[b].at[pl.ds(HALF, HALF)],
                                  acc2.at[ibufs[b].at[1]], ssems[b]).wait()

    issue(w, 0)
    issue(w + NW, 1)

    @pl.loop(0, TSTEPS, step=2)
    def _main(t):
        s0 = w + NW * t
        process(s0, 0)
        issue(s0 + 2 * NW, 0)
        s1 = w + NW * (t + 1)
        process(s1, 1)
        issue(s1 + 2 * NW, 1)

    plsc.subcore_barrier()

    pltpu.sync_copy(acc2.at[pl.ds(s * GROWS, GROWS)], obuf)
    pltpu.sync_copy(accf.at[pl.ds(s * GROWS * D, GROWS * D)], zbuf)

    @pl.loop(0, GROWS)
    def _sum_rows(i):
        @pl.loop(0, NV)
        def _sum_lanes(k):
            off = pl.multiple_of(i * D + k * NLANE, NLANE)
            zbuf[pl.ds(off, NLANE)] = (
                zbuf[pl.ds(off, NLANE)] + obuf[i, pl.ds(k * NLANE, NLANE)])

    pltpu.sync_copy(zbuf, part_hbm.at[c, pl.ds(s * GROWS * D, GROWS * D)])


def _tc_pool(x_ref, ids_ref, o_ref):
    i = pl.program_id(0)

    @pl.when(i == 0)
    def _init():
        o_ref[...] = jnp.zeros_like(o_ref)

    base = (TCX0 + i) * TCBLK
    rows_r = base + lax.broadcasted_iota(jnp.int32, (1, TCBLK), 1)
    ids_eff = jnp.where(rows_r < N, ids_ref[0], G)
    oht = (lax.broadcasted_iota(jnp.int32, (G, TCBLK), 0)
           == ids_eff).astype(jnp.bfloat16)
    rows_c = base + lax.broadcasted_iota(jnp.int32, (TCBLK, 1), 0)
    xb = jnp.where(rows_c < N, x_ref[...], 0.0).astype(jnp.bfloat16)
    o_ref[...] += lax.dot_general(oht, xb, (((1,), (0,)), ((), ())),
                                  preferred_element_type=jnp.float32)


def _tc_add(p_ref, t_ref, o_ref):
    o_ref[...] = p_ref[0] + p_ref[1] + t_ref[...]


@jax.jit
def _pool(x, batch):
    mesh = plsc.VectorSubcoreMesh(core_axis_name="c", subcore_axis_name="s",
                                  num_cores=NC, num_subcores=NS)
    nblk = -(-N // TCBLK)
    batch3d = jnp.pad(batch, (0, nblk * TCBLK - N)).reshape(nblk, 1, TCBLK)
    tc_part = pl.pallas_call(
        _tc_pool,
        grid=(TCGRID,),
        in_specs=[
            pl.BlockSpec((TCBLK, D), lambda i: (TCX0 + i, 0)),
            pl.BlockSpec((1, 1, TCBLK), lambda i: (TCX0 + i, 0, 0)),
        ],
        out_specs=pl.BlockSpec((G, D), lambda i: (0, 0)),
        out_shape=jax.ShapeDtypeStruct((G, D), jnp.float32),
    )(x, batch3d)
    partials = pl.kernel(
        _sc_body,
        out_type=jax.ShapeDtypeStruct((NC, G * D), jnp.float32),
        mesh=mesh,
        scratch_types=[
            pltpu.VMEM((CHUNK, D), jnp.float32),
            pltpu.VMEM((CHUNK, D), jnp.float32),
            pltpu.VMEM((2, IDXW), jnp.int32),
            pltpu.VMEM((2, IDXW), jnp.int32),
            pltpu.VMEM((D,), jnp.float32),
            pltpu.VMEM((D,), jnp.int32),
            pltpu.VMEM((NLANE,), jnp.int32),
            pltpu.VMEM((GROWS, D), jnp.float32),
            pltpu.VMEM((GROWS * D,), jnp.float32),
            pltpu.VMEM_SHARED((G + 8, D), jnp.float32),
            pltpu.VMEM_SHARED(((G + 8) * D,), jnp.float32),
            pltpu.SemaphoreType.DMA,
            pltpu.SemaphoreType.DMA,
            pltpu.SemaphoreType.DMA,
            pltpu.SemaphoreType.DMA,
            pltpu.SemaphoreType.DMA,
            pltpu.SemaphoreType.DMA,
        ],
    )(x, batch)
    summed = pl.pallas_call(
        _tc_add,
        out_shape=jax.ShapeDtypeStruct((G * D,), jnp.float32),
    )(partials, tc_part.reshape(G * D))
    return summed.reshape(G, D)


def kernel(x, batch):
    return _pool(x, batch.astype(jnp.int32))


# speedup vs baseline: 1.6574x; 1.6574x over previous
import jax
import jax.numpy as jnp
from jax import lax
from jax.experimental import pallas as pl
from jax.experimental.pallas import tpu as pltpu
from jax.experimental.pallas import tpu_sc as plsc

N = 100000
D = 128
G = 512
NC = 2
NS = 16
NW = NC * NS
NLANE = 16
NV = D // NLANE
CHUNK = 256
HALF = CHUNK // 2
NGRP = HALF // NLANE
NSLOT = N // CHUNK
TAIL = N - NSLOT * CHUNK
TAILBASE = NSLOT * CHUNK
IDXW = 128
TSTEPS = -(-NSLOT // NW) + 1
GROWS = G // NS


def _sc_body(x_hbm, batch_hbm, part_hbm, xb0, xb1, ib0, ib1,
             obuf, tbuf, tidxa, tidxb, acc2, gs0, gs1, is0, is1):
    c = lax.axis_index("c")
    s = lax.axis_index("s")
    w = s * NC + c
    xbufs, ibufs = (xb0, xb1), (ib0, ib1)
    gsems, isems = (gs0, gs1), (is0, is1)

    @pl.loop(0, GROWS)
    def _zero_rows(i):
        @pl.loop(0, NV)
        def _zero_lanes(k):
            obuf[i, pl.ds(k * NLANE, NLANE)] = jnp.zeros((NLANE,),
                                                         jnp.float32)

    pltpu.sync_copy(obuf, acc2.at[pl.ds(s * GROWS, GROWS)])
    plsc.subcore_barrier()

    def issue(slot, b):
        @pl.when(slot < NSLOT)
        def _():
            row0 = pl.multiple_of(slot * CHUNK, CHUNK)
            pltpu.async_copy(x_hbm.at[pl.ds(row0, CHUNK)], xbufs[b],
                             gsems[b])
            pltpu.async_copy(batch_hbm.at[pl.ds(row0, IDXW)],
                             ibufs[b].at[0], isems[b])
            pltpu.async_copy(batch_hbm.at[pl.ds(row0 + IDXW, IDXW)],
                             ibufs[b].at[1], isems[b])

    def process(slot, b):
        @pl.when(slot < NSLOT)
        def _():
            row0 = pl.multiple_of(slot * CHUNK, CHUNK)
            pltpu.make_async_copy(x_hbm.at[pl.ds(row0, CHUNK)], xbufs[b],
                                  gsems[b]).wait()
            pltpu.make_async_copy(batch_hbm.at[pl.ds(row0, IDXW)],
                                  ibufs[b].at[0], isems[b]).wait()
            pltpu.make_async_copy(batch_hbm.at[pl.ds(row0 + IDXW, IDXW)],
                                  ibufs[b].at[1], isems[b]).wait()

            pltpu.sync_copy(xbufs[b].at[pl.ds(0, HALF)],
                            acc2.at[ibufs[b].at[0]], add=True)
            pltpu.sync_copy(xbufs[b].at[pl.ds(HALF, HALF)],
                            acc2.at[ibufs[b].at[1]], add=True)

    issue(w, 0)
    issue(w + NW, 1)

    @pl.loop(0, TSTEPS, step=2)
    def _main(t):
        s0 = w + NW * t
        process(s0, 0)
        issue(s0 + 2 * NW, 0)
        s1 = w + NW * (t + 1)
        process(s1, 1)
        issue(s1 + 2 * NW, 1)

    @pl.when(w == NW - 1)
    def _tail():
        row0 = pl.multiple_of(TAILBASE, CHUNK)
        pltpu.sync_copy(x_hbm.at[pl.ds(row0, TAIL)], tbuf)
        pltpu.sync_copy(batch_hbm.at[pl.ds(row0, IDXW)], tidxa)
        pltpu.sync_copy(batch_hbm.at[pl.ds(row0 + IDXW, TAIL - IDXW)], tidxb)
        pltpu.sync_copy(tbuf.at[pl.ds(0, IDXW)], acc2.at[tidxa], add=True)
        pltpu.sync_copy(tbuf.at[pl.ds(IDXW, TAIL - IDXW)], acc2.at[tidxb],
                        add=True)

    plsc.subcore_barrier()

    pltpu.sync_copy(acc2.at[pl.ds(s * GROWS, GROWS)], obuf)
    pltpu.sync_copy(obuf, part_hbm.at[c, pl.ds(s * GROWS, GROWS)])


def _tc_add(p_ref, o_ref):
    o_ref[...] = p_ref[0] + p_ref[1]


@jax.jit
def _pool(x, batch):
    mesh = plsc.VectorSubcoreMesh(core_axis_name="c", subcore_axis_name="s",
                                  num_cores=NC, num_subcores=NS)
    partials = pl.kernel(
        _sc_body,
        out_type=jax.ShapeDtypeStruct((NC, G, D), jnp.float32),
        mesh=mesh,
        scratch_types=[
            pltpu.VMEM((CHUNK, D), jnp.float32),
            pltpu.VMEM((CHUNK, D), jnp.float32),
            pltpu.VMEM((2, IDXW), jnp.int32),
            pltpu.VMEM((2, IDXW), jnp.int32),
            pltpu.VMEM((GROWS, D), jnp.float32),
            pltpu.VMEM((TAIL, D), jnp.float32),
            pltpu.VMEM((IDXW,), jnp.int32),
            pltpu.VMEM((TAIL - IDXW,), jnp.int32),
            pltpu.VMEM_SHARED((G + 8, D), jnp.float32),
            pltpu.SemaphoreType.DMA,
            pltpu.SemaphoreType.DMA,
            pltpu.SemaphoreType.DMA,
            pltpu.SemaphoreType.DMA,
        ],
    )(x, batch)
    return pl.pallas_call(
        _tc_add,
        out_shape=jax.ShapeDtypeStruct((G, D), jnp.float32),
    )(partials)


def kernel(x, batch):
    return _pool(x, batch.astype(jnp.int32))


# speedup vs baseline: 1.7723x; 1.0693x over previous
import jax
import jax.numpy as jnp
from jax import lax
from jax.experimental import pallas as pl
from jax.experimental.pallas import tpu as pltpu
from jax.experimental.pallas import tpu_sc as plsc

N = 100000
D = 128
G = 512
NC = 2
NS = 16
NW = NC * NS
NLANE = 16
NV = D // NLANE
CHUNK = 256
HALF = CHUNK // 2
NGRP = HALF // NLANE
NSLOT = 312
P = NSLOT * CHUNK
IDXW = 128
_T = -(-NSLOT // NW)
TSTEPS = _T + (_T % 2)
TCBLK = 512
TCX0 = P // TCBLK
TCGRID = -(-(N - P) // TCBLK)
GROWS = G // NS


def _sc_body(x_hbm, batch_hbm, part_hbm, xb0, xb1, ib0, ib1,
             obuf, acc2, gs0, gs1, is0, is1):
    c = lax.axis_index("c")
    s = lax.axis_index("s")
    w = s * NC + c
    xbufs, ibufs = (xb0, xb1), (ib0, ib1)
    gsems, isems = (gs0, gs1), (is0, is1)

    @pl.loop(0, GROWS)
    def _zero_rows(i):
        @pl.loop(0, NV)
        def _zero_lanes(k):
            obuf[i, pl.ds(k * NLANE, NLANE)] = jnp.zeros((NLANE,),
                                                         jnp.float32)

    pltpu.sync_copy(obuf, acc2.at[pl.ds(s * GROWS, GROWS)])
    plsc.subcore_barrier()

    def issue(slot, b):
        @pl.when(slot < NSLOT)
        def _():
            row0 = pl.multiple_of(slot * CHUNK, CHUNK)
            pltpu.async_copy(x_hbm.at[pl.ds(row0, CHUNK)], xbufs[b],
                             gsems[b])
            pltpu.async_copy(batch_hbm.at[pl.ds(row0, IDXW)],
                             ibufs[b].at[0], isems[b])
            pltpu.async_copy(batch_hbm.at[pl.ds(row0 + IDXW, IDXW)],
                             ibufs[b].at[1], isems[b])

    def process(slot, b):
        @pl.when(slot < NSLOT)
        def _():
            row0 = pl.multiple_of(slot * CHUNK, CHUNK)
            pltpu.make_async_copy(x_hbm.at[pl.ds(row0, CHUNK)], xbufs[b],
                                  gsems[b]).wait()
            pltpu.make_async_copy(batch_hbm.at[pl.ds(row0, IDXW)],
                                  ibufs[b].at[0], isems[b]).wait()
            pltpu.make_async_copy(batch_hbm.at[pl.ds(row0 + IDXW, IDXW)],
                                  ibufs[b].at[1], isems[b]).wait()

            pltpu.sync_copy(xbufs[b].at[pl.ds(0, HALF)],
                            acc2.at[ibufs[b].at[0]], add=True)
            pltpu.sync_copy(xbufs[b].at[pl.ds(HALF, HALF)],
                            acc2.at[ibufs[b].at[1]], add=True)

    issue(w, 0)
    issue(w + NW, 1)

    @pl.loop(0, TSTEPS, step=2)
    def _main(t):
        s0 = w + NW * t
        process(s0, 0)
        issue(s0 + 2 * NW, 0)
        s1 = w + NW * (t + 1)
        process(s1, 1)
        issue(s1 + 2 * NW, 1)

    plsc.subcore_barrier()

    pltpu.sync_copy(acc2.at[pl.ds(s * GROWS, GROWS)], obuf)
    pltpu.sync_copy(obuf, part_hbm.at[c, pl.ds(s * GROWS, GROWS)])


def _tc_pool(x_ref, ids_ref, o_ref):
    i = pl.program_id(0)

    @pl.when(i == 0)
    def _init():
        o_ref[...] = jnp.zeros_like(o_ref)

    base = (TCX0 + i) * TCBLK
    rows_r = base + lax.broadcasted_iota(jnp.int32, (1, TCBLK), 1)
    ids_eff = jnp.where(rows_r < N, ids_ref[0], G)
    oht = (lax.broadcasted_iota(jnp.int32, (G, TCBLK), 0)
           == ids_eff).astype(jnp.bfloat16)
    rows_c = base + lax.broadcasted_iota(jnp.int32, (TCBLK, 1), 0)
    xb = jnp.where(rows_c < N, x_ref[...], 0.0).astype(jnp.bfloat16)
    o_ref[...] += lax.dot_general(oht, xb, (((1,), (0,)), ((), ())),
                                  preferred_element_type=jnp.float32)


def _tc_add(p_ref, t_ref, o_ref):
    o_ref[...] = p_ref[0] + p_ref[1] + t_ref[...]


@jax.jit
def _pool(x, batch):
    mesh = plsc.VectorSubcoreMesh(core_axis_name="c", subcore_axis_name="s",
                                  num_cores=NC, num_subcores=NS)
    nblk = -(-N // TCBLK)
    batch3d = jnp.pad(batch, (0, nblk * TCBLK - N)).reshape(nblk, 1, TCBLK)
    tc_part = pl.pallas_call(
        _tc_pool,
        grid=(TCGRID,),
        in_specs=[
            pl.BlockSpec((TCBLK, D), lambda i: (TCX0 + i, 0)),
            pl.BlockSpec((1, 1, TCBLK), lambda i: (TCX0 + i, 0, 0)),
        ],
        out_specs=pl.BlockSpec((G, D), lambda i: (0, 0)),
        out_shape=jax.ShapeDtypeStruct((G, D), jnp.float32),
    )(x, batch3d)
    partials = pl.kernel(
        _sc_body,
        out_type=jax.ShapeDtypeStruct((NC, G, D), jnp.float32),
        mesh=mesh,
        scratch_types=[
            pltpu.VMEM((CHUNK, D), jnp.float32),
            pltpu.VMEM((CHUNK, D), jnp.float32),
            pltpu.VMEM((2, IDXW), jnp.int32),
            pltpu.VMEM((2, IDXW), jnp.int32),
            pltpu.VMEM((GROWS, D), jnp.float32),
            pltpu.VMEM_SHARED((G + 8, D), jnp.float32),
            pltpu.SemaphoreType.DMA,
            pltpu.SemaphoreType.DMA,
            pltpu.SemaphoreType.DMA,
            pltpu.SemaphoreType.DMA,
        ],
    )(x, batch)
    return pl.pallas_call(
        _tc_add,
        out_shape=jax.ShapeDtypeStruct((G, D), jnp.float32),
    )(partials, tc_part)


def kernel(x, batch):
    return _pool(x, batch.astype(jnp.int32))


# speedup vs baseline: 1.8645x; 1.0520x over previous
import jax
import jax.numpy as jnp
from jax import lax
from jax.experimental import pallas as pl
from jax.experimental.pallas import tpu as pltpu
from jax.experimental.pallas import tpu_sc as plsc

N = 100000
D = 128
G = 512
NC = 2
NS = 16
NW = NC * NS
NLANE = 16
NV = D // NLANE
CHUNK = 256
HALF = CHUNK // 2
NGRP = HALF // NLANE
NSLOT = 320
P = NSLOT * CHUNK
IDXW = 128
_T = -(-NSLOT // NW)
TSTEPS = _T + (_T % 2)
TCBLK = 512
TCX0 = P // TCBLK
TCGRID = -(-(N - P) // TCBLK)
GROWS = G // NS


def _sc_body(x_hbm, batch_hbm, part_hbm, xb0, xb1, ib0, ib1,
             obuf, acc2, gs0, gs1, is0, is1):
    c = lax.axis_index("c")
    s = lax.axis_index("s")
    w = s * NC + c
    xbufs, ibufs = (xb0, xb1), (ib0, ib1)
    gsems, isems = (gs0, gs1), (is0, is1)

    @pl.loop(0, GROWS)
    def _zero_rows(i):
        @pl.loop(0, NV)
        def _zero_lanes(k):
            obuf[i, pl.ds(k * NLANE, NLANE)] = jnp.zeros((NLANE,),
                                                         jnp.float32)

    pltpu.sync_copy(obuf, acc2.at[pl.ds(s * GROWS, GROWS)])
    plsc.subcore_barrier()

    def issue(slot, b):
        @pl.when(slot < NSLOT)
        def _():
            row0 = pl.multiple_of(slot * CHUNK, CHUNK)
            pltpu.async_copy(x_hbm.at[pl.ds(row0, CHUNK)], xbufs[b],
                             gsems[b])
            pltpu.async_copy(batch_hbm.at[pl.ds(row0, IDXW)],
                             ibufs[b].at[0], isems[b])
            pltpu.async_copy(batch_hbm.at[pl.ds(row0 + IDXW, IDXW)],
                             ibufs[b].at[1], isems[b])

    def process(slot, b):
        @pl.when(slot < NSLOT)
        def _():
            row0 = pl.multiple_of(slot * CHUNK, CHUNK)
            pltpu.make_async_copy(x_hbm.at[pl.ds(row0, CHUNK)], xbufs[b],
                                  gsems[b]).wait()
            pltpu.make_async_copy(batch_hbm.at[pl.ds(row0, IDXW)],
                                  ibufs[b].at[0], isems[b]).wait()
            pltpu.make_async_copy(batch_hbm.at[pl.ds(row0 + IDXW, IDXW)],
                                  ibufs[b].at[1], isems[b]).wait()

            pltpu.sync_copy(xbufs[b].at[pl.ds(0, HALF)],
                            acc2.at[ibufs[b].at[0]], add=True)
            pltpu.sync_copy(xbufs[b].at[pl.ds(HALF, HALF)],
                            acc2.at[ibufs[b].at[1]], add=True)

    issue(w, 0)
    issue(w + NW, 1)

    @pl.loop(0, TSTEPS, step=2)
    def _main(t):
        s0 = w + NW * t
        process(s0, 0)
        issue(s0 + 2 * NW, 0)
        s1 = w + NW * (t + 1)
        process(s1, 1)
        issue(s1 + 2 * NW, 1)

    plsc.subcore_barrier()

    pltpu.sync_copy(acc2.at[pl.ds(s * GROWS, GROWS)], obuf)
    pltpu.sync_copy(obuf, part_hbm.at[c, pl.ds(s * GROWS, GROWS)])


def _tc_pool(x_ref, ids_ref, o_ref):
    i = pl.program_id(0)

    @pl.when(i == 0)
    def _init():
        o_ref[...] = jnp.zeros_like(o_ref)

    base = (TCX0 + i) * TCBLK
    rows_r = base + lax.broadcasted_iota(jnp.int32, (1, TCBLK), 1)
    ids_eff = jnp.where(rows_r < N, ids_ref[0], G)
    oht = (lax.broadcasted_iota(jnp.int32, (G, TCBLK), 0)
           == ids_eff).astype(jnp.bfloat16)
    rows_c = base + lax.broadcasted_iota(jnp.int32, (TCBLK, 1), 0)
    xb = jnp.where(rows_c < N, x_ref[...], 0.0).astype(jnp.bfloat16)
    o_ref[...] += lax.dot_general(oht, xb, (((1,), (0,)), ((), ())),
                                  preferred_element_type=jnp.float32)


def _tc_add(p_ref, t_ref, o_ref):
    o_ref[...] = p_ref[0] + p_ref[1] + t_ref[...]


@jax.jit
def _pool(x, batch):
    mesh = plsc.VectorSubcoreMesh(core_axis_name="c", subcore_axis_name="s",
                                  num_cores=NC, num_subcores=NS)
    nblk = -(-N // TCBLK)
    batch3d = jnp.pad(batch, (0, nblk * TCBLK - N)).reshape(nblk, 1, TCBLK)
    tc_part = pl.pallas_call(
        _tc_pool,
        grid=(TCGRID,),
        in_specs=[
            pl.BlockSpec((TCBLK, D), lambda i: (TCX0 + i, 0)),
            pl.BlockSpec((1, 1, TCBLK), lambda i: (TCX0 + i, 0, 0)),
        ],
        out_specs=pl.BlockSpec((G, D), lambda i: (0, 0)),
        out_shape=jax.ShapeDtypeStruct((G, D), jnp.float32),
    )(x, batch3d)
    partials = pl.kernel(
        _sc_body,
        out_type=jax.ShapeDtypeStruct((NC, G, D), jnp.float32),
        mesh=mesh,
        scratch_types=[
            pltpu.VMEM((CHUNK, D), jnp.float32),
            pltpu.VMEM((CHUNK, D), jnp.float32),
            pltpu.VMEM((2, IDXW), jnp.int32),
            pltpu.VMEM((2, IDXW), jnp.int32),
            pltpu.VMEM((GROWS, D), jnp.float32),
            pltpu.VMEM_SHARED((G + 8, D), jnp.float32),
            pltpu.SemaphoreType.DMA,
            pltpu.SemaphoreType.DMA,
            pltpu.SemaphoreType.DMA,
            pltpu.SemaphoreType.DMA,
        ],
    )(x, batch)
    return pl.pallas_call(
        _tc_add,
        out_shape=jax.ShapeDtypeStruct((G, D), jnp.float32),
    )(partials, tc_part)


def kernel(x, batch):
    return _pool(x, batch.astype(jnp.int32))


# speedup vs baseline: 1.9433x; 1.0423x over previous
import jax
import jax.numpy as jnp
from jax import lax
from jax.experimental import pallas as pl
from jax.experimental.pallas import tpu as pltpu
from jax.experimental.pallas import tpu_sc as plsc

N = 100000
D = 128
G = 512
NC = 2
NS = 16
NW = NC * NS
NLANE = 16
NV = D // NLANE
CHUNK = 256
HALF = CHUNK // 2
NGRP = HALF // NLANE
NSLOT = 288
P = NSLOT * CHUNK
IDXW = 128
_T = -(-NSLOT // NW)
TSTEPS = _T + (_T % 2)
TCBLK = 1024
TCX0 = P // TCBLK
TCGRID = -(-(N - P) // TCBLK)
GROWS = G // NS


def _sc_body(x_hbm, batch_hbm, part_hbm, xb0, xb1, ib0, ib1,
             obuf, acc2, gs0, gs1, is0, is1):
    c = lax.axis_index("c")
    s = lax.axis_index("s")
    w = s * NC + c
    xbufs, ibufs = (xb0, xb1), (ib0, ib1)
    gsems, isems = (gs0, gs1), (is0, is1)

    @pl.loop(0, GROWS)
    def _zero_rows(i):
        @pl.loop(0, NV)
        def _zero_lanes(k):
            obuf[i, pl.ds(k * NLANE, NLANE)] = jnp.zeros((NLANE,),
                                                         jnp.float32)

    pltpu.sync_copy(obuf, acc2.at[pl.ds(s * GROWS, GROWS)])
    plsc.subcore_barrier()

    def issue(slot, b):
        @pl.when(slot < NSLOT)
        def _():
            row0 = pl.multiple_of(slot * CHUNK, CHUNK)
            pltpu.async_copy(x_hbm.at[pl.ds(row0, CHUNK)], xbufs[b],
                             gsems[b])
            pltpu.async_copy(batch_hbm.at[pl.ds(row0, IDXW)],
                             ibufs[b].at[0], isems[b])
            pltpu.async_copy(batch_hbm.at[pl.ds(row0 + IDXW, IDXW)],
                             ibufs[b].at[1], isems[b])

    def process(slot, b):
        @pl.when(slot < NSLOT)
        def _():
            row0 = pl.multiple_of(slot * CHUNK, CHUNK)
            pltpu.make_async_copy(x_hbm.at[pl.ds(row0, CHUNK)], xbufs[b],
                                  gsems[b]).wait()
            pltpu.make_async_copy(batch_hbm.at[pl.ds(row0, IDXW)],
                                  ibufs[b].at[0], isems[b]).wait()
            pltpu.make_async_copy(batch_hbm.at[pl.ds(row0 + IDXW, IDXW)],
                                  ibufs[b].at[1], isems[b]).wait()

            pltpu.sync_copy(xbufs[b].at[pl.ds(0, HALF)],
                            acc2.at[ibufs[b].at[0]], add=True)
            pltpu.sync_copy(xbufs[b].at[pl.ds(HALF, HALF)],
                            acc2.at[ibufs[b].at[1]], add=True)

    issue(w, 0)
    issue(w + NW, 1)

    @pl.loop(0, TSTEPS, step=2)
    def _main(t):
        s0 = w + NW * t
        process(s0, 0)
        issue(s0 + 2 * NW, 0)
        s1 = w + NW * (t + 1)
        process(s1, 1)
        issue(s1 + 2 * NW, 1)

    plsc.subcore_barrier()

    pltpu.sync_copy(acc2.at[pl.ds(s * GROWS, GROWS)], obuf)
    pltpu.sync_copy(obuf, part_hbm.at[c, pl.ds(s * GROWS, GROWS)])


def _tc_pool(x_ref, ids_ref, o_ref):
    i = pl.program_id(0)

    @pl.when(i == 0)
    def _init():
        o_ref[...] = jnp.zeros_like(o_ref)

    base = (TCX0 + i) * TCBLK
    rows_r = base + lax.broadcasted_iota(jnp.int32, (1, TCBLK), 1)
    ids_eff = jnp.where(rows_r < N, ids_ref[0], G)
    oht = (lax.broadcasted_iota(jnp.int32, (G, TCBLK), 0)
           == ids_eff).astype(jnp.bfloat16)
    rows_c = base + lax.broadcasted_iota(jnp.int32, (TCBLK, 1), 0)
    xb = jnp.where(rows_c < N, x_ref[...], 0.0).astype(jnp.bfloat16)
    o_ref[...] += lax.dot_general(oht, xb, (((1,), (0,)), ((), ())),
                                  preferred_element_type=jnp.float32)


def _tc_add(p_ref, t_ref, o_ref):
    o_ref[...] = p_ref[0] + p_ref[1] + t_ref[...]


@jax.jit
def _pool(x, batch):
    mesh = plsc.VectorSubcoreMesh(core_axis_name="c", subcore_axis_name="s",
                                  num_cores=NC, num_subcores=NS)
    nblk = -(-N // TCBLK)
    batch3d = jnp.pad(batch, (0, nblk * TCBLK - N)).reshape(nblk, 1, TCBLK)
    tc_part = pl.pallas_call(
        _tc_pool,
        grid=(TCGRID,),
        in_specs=[
            pl.BlockSpec((TCBLK, D), lambda i: (TCX0 + i, 0)),
            pl.BlockSpec((1, 1, TCBLK), lambda i: (TCX0 + i, 0, 0)),
        ],
        out_specs=pl.BlockSpec((G, D), lambda i: (0, 0)),
        out_shape=jax.ShapeDtypeStruct((G, D), jnp.float32),
    )(x, batch3d)
    partials = pl.kernel(
        _sc_body,
        out_type=jax.ShapeDtypeStruct((NC, G, D), jnp.float32),
        mesh=mesh,
        scratch_types=[
            pltpu.VMEM((CHUNK, D), jnp.float32),
            pltpu.VMEM((CHUNK, D), jnp.float32),
            pltpu.VMEM((2, IDXW), jnp.int32),
            pltpu.VMEM((2, IDXW), jnp.int32),
            pltpu.VMEM((GROWS, D), jnp.float32),
            pltpu.VMEM_SHARED((G + 8, D), jnp.float32),
            pltpu.SemaphoreType.DMA,
            pltpu.SemaphoreType.DMA,
            pltpu.SemaphoreType.DMA,
            pltpu.SemaphoreType.DMA,
        ],
    )(x, batch)
    return pl.pallas_call(
        _tc_add,
        out_shape=jax.ShapeDtypeStruct((G, D), jnp.float32),
    )(partials, tc_part)


def kernel(x, batch):
    return _pool(x, batch.astype(jnp.int32))
